# Initial kernel scaffold; baseline (speedup 1.0000x reference)
import functools

import jax
import jax.numpy as jnp
from jax import lax
from jax.experimental import pallas as pl
from jax.experimental.pallas import tpu as pltpu
from jax.experimental.pallas import tpu_sc as plsc

B, N, M, C = 4, 8192, 2048, 128
NS = 32
RADIUS2 = 0.2 * 0.2
NC, NSUB, L = 2, 16, 16
NTILE = NC * NSUB
QT = (B * M) // NTILE
HITCAP = 160
CHUNK = 8


def _mlp_body(f_ref, x_ref, w1_ref, b1_ref, w2_ref, b2_ref, wx_ref, t_ref):
    f = f_ref[0]
    h = jnp.maximum(
        jnp.dot(f, w1_ref[...], preferred_element_type=jnp.float32)
        + b1_ref[...], 0.0)
    h = jnp.maximum(
        jnp.dot(h, w2_ref[...], preferred_element_type=jnp.float32)
        + b2_ref[...], 0.0)
    u = jnp.dot(x_ref[0], wx_ref[...],
                preferred_element_type=jnp.float32)
    t_ref[0, :, :256] = h
    t_ref[0, :, 256:288] = u


def _mlp_table(feats_t, xyz_pad, w1t, b1r, w2t, b2r, wxt):
    bn = 512
    return pl.pallas_call(
        _mlp_body,
        grid=(B, N // bn),
        in_specs=[
            pl.BlockSpec((1, bn, C), lambda b, i: (b, i, 0)),
            pl.BlockSpec((1, bn, 4), lambda b, i: (b, i, 0)),
            pl.BlockSpec((C, 256), lambda b, i: (0, 0)),
            pl.BlockSpec((1, 256), lambda b, i: (0, 0)),
            pl.BlockSpec((256, 256), lambda b, i: (0, 0)),
            pl.BlockSpec((1, 256), lambda b, i: (0, 0)),
            pl.BlockSpec((4, 32), lambda b, i: (0, 0)),
        ],
        out_specs=pl.BlockSpec((1, bn, 288), lambda b, i: (b, i, 0)),
        out_shape=jax.ShapeDtypeStruct((B, N, 288), jnp.float32),
    )(feats_t, xyz_pad, w1t, b1r, w2t, b2r, wxt)


def _newton_rsqrt(s):
    i = plsc.bitcast(s, jnp.int32)
    y = plsc.bitcast(jnp.int32(0x5F3759DF) - (i >> 1), jnp.float32)
    for _ in range(3):
        y = y * (1.5 - 0.5 * s * y * y)
    return y


def _ballq_body(xyz_hbm, new_hbm, idx_hbm, w_hbm,
                xv, yv, zv, qxv, qyv, qzv, ibuf, dbuf, oidx, ow):
    wid = lax.axis_index("s") * NC + lax.axis_index("c")
    q0 = wid * QT
    b = q0 // M
    m0 = q0 - b * M
    pltpu.sync_copy(xyz_hbm.at[b, 0], xv)
    pltpu.sync_copy(xyz_hbm.at[b, 1], yv)
    pltpu.sync_copy(xyz_hbm.at[b, 2], zv)
    pltpu.sync_copy(new_hbm.at[b, 0, pl.ds(m0, QT)], qxv)
    pltpu.sync_copy(new_hbm.at[b, 1, pl.ds(m0, QT)], qyv)
    pltpu.sync_copy(new_hbm.at[b, 2, pl.ds(m0, QT)], qzv)

    lane = lax.iota(jnp.int32, L)
    zeros16 = jnp.zeros((L,), jnp.int32)

    @pl.loop(0, QT)
    def _per_query(j):
        jj = jnp.full((L,), j, jnp.int32)
        qx = plsc.load_gather(qxv, [jj])
        qy = plsc.load_gather(qyv, [jj])
        qz = plsc.load_gather(qzv, [jj])

        def cond(carry):
            n, cnt = carry
            return (n < N) & jnp.all(cnt < NS)

        def body(carry):
            n, cnt = carry
            for v in range(CHUNK):
                base = n + v * L
                dx = xv[pl.ds(base, L)] - qx
                dy = yv[pl.ds(base, L)] - qy
                dz = zv[pl.ds(base, L)] - qz
                d2 = dx * dx + dy * dy + dz * dz
                hit = d2 < RADIUS2
                pos = cnt + plsc.cumsum(hit.astype(jnp.int32)) - 1
                plsc.store_scatter(ibuf, [pos], base + lane, mask=hit)
                plsc.store_scatter(dbuf, [pos], d2, mask=hit)
                cnt = cnt + plsc.all_reduce_population_count(hit)
            return n + CHUNK * L, cnt

        _, cnt_vec = lax.while_loop(cond, body, (jnp.int32(0), zeros16))
        cnt = jnp.minimum(jnp.min(cnt_vec), NS)
        cntv = jnp.full((L,), cnt, jnp.int32)
        nonempty = cntv > 0
        dupf = (NS + 1 - cntv).astype(jnp.float32)
        first_i = plsc.load_gather(ibuf, [zeros16])
        first_d = plsc.load_gather(dbuf, [zeros16])
        rs_tot = jnp.zeros((L,), jnp.float32)
        halves = []
        for h in range(2):
            k = lane + h * L
            valid = k < cntv
            idx_h = jnp.where(valid, ibuf[pl.ds(h * L, L)], first_i)
            idx_h = jnp.where(nonempty, idx_h, 0)
            d2_h = jnp.where(valid, dbuf[pl.ds(h * L, L)], first_d)
            d2_h = jnp.where(nonempty, d2_h, 1.0)
            t = _newton_rsqrt(d2_h + 1e-12)
            r = t / (1.0 + 1e-8 * t)
            idn = jnp.where((k >= 1) & valid, 1.0, dupf)
            r = r / idn
            rs_tot = rs_tot + r
            halves.append((idx_h, r))
        norm = jnp.sum(rs_tot)
        for h, (idx_h, r) in enumerate(halves):
            w = jnp.where(nonempty, r / norm, 0.0)
            oidx[j, pl.ds(h * L, L)] = idx_h + b * N
            ow[j, pl.ds(h * L, L)] = w

    pltpu.sync_copy(oidx, idx_hbm.at[pl.ds(q0, QT)])
    pltpu.sync_copy(ow, w_hbm.at[pl.ds(q0, QT)])


def _ballq(xyz_t, new_t):
    mesh = plsc.VectorSubcoreMesh(core_axis_name="c", subcore_axis_name="s")
    kern = pl.kernel(
        _ballq_body,
        out_type=(
            jax.ShapeDtypeStruct((B * M, NS), jnp.int32),
            jax.ShapeDtypeStruct((B * M, NS), jnp.float32),
        ),
        mesh=mesh,
        scratch_types=[
            pltpu.VMEM((N,), jnp.float32),
            pltpu.VMEM((N,), jnp.float32),
            pltpu.VMEM((N,), jnp.float32),
            pltpu.VMEM((QT,), jnp.float32),
            pltpu.VMEM((QT,), jnp.float32),
            pltpu.VMEM((QT,), jnp.float32),
            pltpu.VMEM((HITCAP,), jnp.int32),
            pltpu.VMEM((HITCAP,), jnp.float32),
            pltpu.VMEM((QT, NS), jnp.int32),
            pltpu.VMEM((QT, NS), jnp.float32),
        ],
    )
    return kern(xyz_t, new_t)


def _agg_body(t_hbm, idx_hbm, w_hbm, nfu_hbm, sidx, sw, hbuf, onf):
    wid = lax.axis_index("s") * NC + lax.axis_index("c")
    q0 = wid * QT
    pltpu.sync_copy(idx_hbm.at[pl.ds(q0, QT)], sidx)
    pltpu.sync_copy(w_hbm.at[pl.ds(q0, QT)], sw)

    @pl.loop(0, QT)
    def _per_query(j):
        pltpu.sync_copy(t_hbm.at[sidx.at[j]], hbuf)
        jj = jnp.full((L,), j, jnp.int32)
        accs = [jnp.zeros((L,), jnp.float32) for _ in range(16)]
        u0 = jnp.full((L,), -jnp.inf, jnp.float32)
        u1 = jnp.full((L,), -jnp.inf, jnp.float32)
        for k in range(NS):
            wk = plsc.load_gather(sw, [jj, jnp.full((L,), k, jnp.int32)])
            for c in range(16):
                accs[c] = accs[c] + wk * hbuf[k, pl.ds(c * L, L)]
            u0 = jnp.maximum(u0, hbuf[k, pl.ds(256, L)])
            u1 = jnp.maximum(u1, hbuf[k, pl.ds(256 + L, L)])
        for c in range(16):
            onf[j, pl.ds(c * L, L)] = accs[c]
        onf[j, pl.ds(256, L)] = u0
        onf[j, pl.ds(256 + L, L)] = u1

    pltpu.sync_copy(onf, nfu_hbm.at[pl.ds(q0, QT)])


def _aggregate(table, idxg, wsel):
    mesh = plsc.VectorSubcoreMesh(core_axis_name="c", subcore_axis_name="s")
    kern = pl.kernel(
        _agg_body,
        out_type=jax.ShapeDtypeStruct((B * M, 288), jnp.float32),
        mesh=mesh,
        scratch_types=[
            pltpu.VMEM((QT, NS), jnp.int32),
            pltpu.VMEM((QT, NS), jnp.float32),
            pltpu.VMEM((NS, 288), jnp.float32),
            pltpu.VMEM((QT, 288), jnp.float32),
        ],
    )
    return kern(table, idxg, wsel)


def _asm_body(nfu_ref, q_ref, wx_ref, bx_ref, o_ref):
    x = nfu_ref[0]
    v = bx_ref[...] - jnp.dot(q_ref[0], wx_ref[...],
                              preferred_element_type=jnp.float32)
    hx = jnp.maximum(x[:, 256:288] + v, 0.0)
    o_ref[0, :32, :] = hx.T
    o_ref[0, 32:, :] = x[:, :256].T


def _assemble(nfu, new_pad, wxt, bxr):
    bm = 512
    return pl.pallas_call(
        _asm_body,
        grid=(B, M // bm),
        in_specs=[
            pl.BlockSpec((1, bm, 288), lambda b, i: (b, i, 0)),
            pl.BlockSpec((1, bm, 4), lambda b, i: (b, i, 0)),
            pl.BlockSpec((4, 32), lambda b, i: (0, 0)),
            pl.BlockSpec((1, 32), lambda b, i: (0, 0)),
        ],
        out_specs=pl.BlockSpec((1, 288, bm), lambda b, i: (b, 0, i)),
        out_shape=jax.ShapeDtypeStruct((B, 288, M), jnp.float32),
    )(nfu, new_pad, wxt, bxr)


def kernel(xyz, new_xyz, features, W1, b1, W2, b2, Wxyz, bxyz):
    feats_t = features.transpose(0, 2, 1)
    xyz_pad = jnp.pad(xyz, ((0, 0), (0, 0), (0, 1)))
    new_pad = jnp.pad(new_xyz, ((0, 0), (0, 0), (0, 1)))
    xyz_t = xyz.transpose(0, 2, 1)
    new_t = new_xyz.transpose(0, 2, 1)
    wxt = jnp.pad(Wxyz.T, ((0, 1), (0, 0)))

    table = _mlp_table(feats_t, xyz_pad, W1.T, b1[None], W2.T, b2[None], wxt)
    idxg, wsel = _ballq(xyz_t, new_t)
    nfu = _aggregate(table.reshape(B * N, 288), idxg, wsel)
    out = _assemble(nfu.reshape(B, M, 288), new_pad, wxt, bxyz[None])
    return (new_xyz, out)


# speedup vs baseline: 17.5929x; 17.5929x over previous
import dataclasses
import functools

import jax
import jax.numpy as jnp
from jax import lax
from jax.experimental import pallas as pl
from jax.experimental.pallas import tpu as pltpu
from jax.experimental.pallas import tpu_sc as plsc

B, N, M, C = 4, 8192, 2048, 128
NS = 32
RADIUS2 = 0.2 * 0.2
NC, NSUB, L = 2, 16, 16
NTILE = NC * NSUB
QT = (B * M) // NTILE
TW = 384
HITCAP = 160
CHUNK = 8


def _sc_compiler_params():
    cp = pltpu.CompilerParams()
    if "needs_layout_passes" in pltpu.CompilerParams.__dataclass_fields__:
        cp = dataclasses.replace(cp, needs_layout_passes=False)
    return cp


def _mlp_body(f_ref, x_ref, w1_ref, b1_ref, w2_ref, b2_ref, wx_ref, t_ref):
    f = f_ref[0]
    h = jnp.maximum(
        jnp.dot(f, w1_ref[...], preferred_element_type=jnp.float32,
                precision=lax.Precision.HIGHEST)
        + b1_ref[...], 0.0)
    h = jnp.maximum(
        jnp.dot(h, w2_ref[...], preferred_element_type=jnp.float32,
                precision=lax.Precision.HIGHEST)
        + b2_ref[...], 0.0)
    u = jnp.dot(x_ref[0], wx_ref[...], preferred_element_type=jnp.float32,
                precision=lax.Precision.HIGHEST)
    t_ref[0, :, :256] = h
    t_ref[0, :, 256:288] = u
    t_ref[0, :, 288:] = jnp.zeros_like(t_ref[0, :, 288:])


def _mlp_table(feats_t, xyz_pad, w1t, b1r, w2t, b2r, wxt):
    bn = 512
    return pl.pallas_call(
        _mlp_body,
        grid=(B, N // bn),
        in_specs=[
            pl.BlockSpec((1, bn, C), lambda b, i: (b, i, 0)),
            pl.BlockSpec((1, bn, 4), lambda b, i: (b, i, 0)),
            pl.BlockSpec((C, 256), lambda b, i: (0, 0)),
            pl.BlockSpec((1, 256), lambda b, i: (0, 0)),
            pl.BlockSpec((256, 256), lambda b, i: (0, 0)),
            pl.BlockSpec((1, 256), lambda b, i: (0, 0)),
            pl.BlockSpec((4, 32), lambda b, i: (0, 0)),
        ],
        out_specs=pl.BlockSpec((1, bn, TW), lambda b, i: (b, i, 0)),
        out_shape=jax.ShapeDtypeStruct((B, N, TW), jnp.float32),
    )(feats_t, xyz_pad, w1t, b1r, w2t, b2r, wxt)


def _bf16_rne(x):
    i = plsc.bitcast(x, jnp.int32)
    r = (i + jnp.int32(0x7FFF) + ((i >> 16) & 1)) & jnp.int32(-65536)
    return plsc.bitcast(r, jnp.float32)


def _newton_rsqrt(s):
    i = plsc.bitcast(s, jnp.int32)
    y = plsc.bitcast(jnp.int32(0x5F3759DF) - (i >> 1), jnp.float32)
    for _ in range(3):
        y = y * (1.5 - 0.5 * s * y * y)
    return y


def _ballq_body(xyz_hbm, new_hbm, idx_hbm, w_hbm,
                xv, yv, zv, qxv, qyv, qzv, ibuf, dbuf, oidx, ow):
    wid = lax.axis_index("s") * NC + lax.axis_index("c")
    q0 = wid * QT
    b = q0 // M
    m0 = q0 - b * M
    pltpu.sync_copy(xyz_hbm.at[pl.ds((b * 3 + 0) * N, N)], xv)
    pltpu.sync_copy(xyz_hbm.at[pl.ds((b * 3 + 1) * N, N)], yv)
    pltpu.sync_copy(xyz_hbm.at[pl.ds((b * 3 + 2) * N, N)], zv)
    pltpu.sync_copy(new_hbm.at[pl.ds((b * 3 + 0) * M + m0, QT)], qxv)
    pltpu.sync_copy(new_hbm.at[pl.ds((b * 3 + 1) * M + m0, QT)], qyv)
    pltpu.sync_copy(new_hbm.at[pl.ds((b * 3 + 2) * M + m0, QT)], qzv)

    lane = lax.iota(jnp.int32, L)
    zeros16 = jnp.zeros((L,), jnp.int32)

    @pl.loop(0, QT)
    def _per_query(j):
        jj = jnp.full((L,), j, jnp.int32)
        qx = plsc.load_gather(qxv, [jj])
        qy = plsc.load_gather(qyv, [jj])
        qz = plsc.load_gather(qzv, [jj])
        q2 = qx * qx + qy * qy + qz * qz
        qxb = _bf16_rne(qx)
        qyb = _bf16_rne(qy)
        qzb = _bf16_rne(qz)

        def cond(carry):
            n, cnt = carry
            return (n < N) & jnp.all(cnt < NS)

        def body(carry):
            n, cnt = carry
            for v in range(CHUNK):
                base = n + v * L
                x = xv[pl.ds(base, L)]
                y = yv[pl.ds(base, L)]
                z = zv[pl.ds(base, L)]
                dx = x - qx
                dy = y - qy
                dz = z - qz
                d2 = dx * dx + dy * dy + dz * dz
                x2 = x * x + y * y + z * z
                dot = qxb * _bf16_rne(x) + qyb * _bf16_rne(y) \
                    + qzb * _bf16_rne(z)
                d2s = (q2 + x2) - 2.0 * dot
                hit = d2s < RADIUS2
                pos = cnt + plsc.cumsum(hit.astype(jnp.int32)) - 1
                plsc.store_scatter(ibuf, [pos], base + lane, mask=hit)
                plsc.store_scatter(dbuf, [pos], d2, mask=hit)
                cnt = cnt + plsc.all_reduce_population_count(hit)
            return n + CHUNK * L, cnt

        _, cnt_vec = lax.while_loop(cond, body, (jnp.int32(0), zeros16))
        cnt = jnp.minimum(jnp.min(cnt_vec), NS)
        cntv = jnp.full((L,), cnt, jnp.int32)
        nonempty = cntv > 0
        dupf = (NS + 1 - cntv).astype(jnp.float32)
        first_i = plsc.load_gather(ibuf, [zeros16])
        first_d = plsc.load_gather(dbuf, [zeros16])
        rs_tot = jnp.zeros((L,), jnp.float32)
        halves = []
        for h in range(2):
            k = lane + h * L
            valid = k < cntv
            idx_h = jnp.where(valid, ibuf[pl.ds(h * L, L)], first_i)
            idx_h = jnp.where(nonempty, idx_h, 0)
            d2_h = jnp.where(valid, dbuf[pl.ds(h * L, L)], first_d)
            d2_h = jnp.where(nonempty, d2_h, 1.0)
            t = _newton_rsqrt(d2_h + 1e-12)
            r = t / (1.0 + 1e-8 * t)
            idn = jnp.where((k >= 1) & valid, 1.0, dupf)
            r = r / idn
            rs_tot = rs_tot + r
            halves.append((idx_h, r))
        norm = jnp.sum(rs_tot)
        for h, (idx_h, r) in enumerate(halves):
            w = jnp.where(nonempty, r / norm, 0.0)
            oidx[pl.ds(j * NS + h * L, L)] = idx_h + b * N
            ow[pl.ds(j * NS + h * L, L)] = w

    pltpu.sync_copy(oidx, idx_hbm.at[pl.ds(q0 * NS, QT * NS)])
    pltpu.sync_copy(ow, w_hbm.at[pl.ds(q0 * NS, QT * NS)])


def _ballq(xyz_t, new_t):
    mesh = plsc.VectorSubcoreMesh(core_axis_name="c", subcore_axis_name="s")
    kern = pl.kernel(
        _ballq_body,
        out_type=(
            jax.ShapeDtypeStruct((B * M * NS,), jnp.int32),
            jax.ShapeDtypeStruct((B * M * NS,), jnp.float32),
        ),
        mesh=mesh,
        compiler_params=_sc_compiler_params(),
        scratch_types=[
            pltpu.VMEM((N,), jnp.float32),
            pltpu.VMEM((N,), jnp.float32),
            pltpu.VMEM((N,), jnp.float32),
            pltpu.VMEM((QT,), jnp.float32),
            pltpu.VMEM((QT,), jnp.float32),
            pltpu.VMEM((QT,), jnp.float32),
            pltpu.VMEM((HITCAP,), jnp.int32),
            pltpu.VMEM((HITCAP,), jnp.float32),
            pltpu.VMEM((QT * NS,), jnp.int32),
            pltpu.VMEM((QT * NS,), jnp.float32),
        ],
    )
    return kern(xyz_t, new_t)


def _agg_body(t_hbm, idx_hbm, w_hbm, nfu_hbm, sidx, sw, hbuf, onf):
    wid = lax.axis_index("s") * NC + lax.axis_index("c")
    q0 = wid * QT
    pltpu.sync_copy(idx_hbm.at[pl.ds(q0 * NS, QT * NS)], sidx)
    pltpu.sync_copy(w_hbm.at[pl.ds(q0 * NS, QT * NS)], sw)

    @pl.loop(0, QT)
    def _per_query(j):
        pltpu.sync_copy(t_hbm.at[sidx.at[pl.ds(j * NS, NS)]], hbuf)
        accs = [jnp.zeros((L,), jnp.float32) for _ in range(16)]
        u0 = jnp.full((L,), -jnp.inf, jnp.float32)
        u1 = jnp.full((L,), -jnp.inf, jnp.float32)
        for k in range(NS):
            wk = plsc.load_gather(sw, [jnp.full((L,), j * NS + k, jnp.int32)])
            for c in range(16):
                accs[c] = accs[c] + wk * hbuf[k, pl.ds(c * L, L)]
            u0 = jnp.maximum(u0, hbuf[k, pl.ds(256, L)])
            u1 = jnp.maximum(u1, hbuf[k, pl.ds(256 + L, L)])
        for c in range(16):
            onf[pl.ds(j * 288 + c * L, L)] = accs[c]
        onf[pl.ds(j * 288 + 256, L)] = u0
        onf[pl.ds(j * 288 + 256 + L, L)] = u1

    pltpu.sync_copy(onf, nfu_hbm.at[pl.ds(q0 * 288, QT * 288)])


def _aggregate(table, idxg, wsel):
    mesh = plsc.VectorSubcoreMesh(core_axis_name="c", subcore_axis_name="s")
    kern = pl.kernel(
        _agg_body,
        out_type=jax.ShapeDtypeStruct((B * M * 288,), jnp.float32),
        mesh=mesh,
        compiler_params=_sc_compiler_params(),
        scratch_types=[
            pltpu.VMEM((QT * NS,), jnp.int32),
            pltpu.VMEM((QT * NS,), jnp.float32),
            pltpu.VMEM((NS, TW), jnp.float32),
            pltpu.VMEM((QT * 288,), jnp.float32),
        ],
    )
    return kern(table, idxg, wsel)


def _asm_body(nfu_ref, q_ref, wx_ref, bx_ref, o_ref):
    x = nfu_ref[0]
    v = bx_ref[...] - jnp.dot(q_ref[0], wx_ref[...],
                              preferred_element_type=jnp.float32,
                              precision=lax.Precision.HIGHEST)
    hx = jnp.maximum(x[:, 256:288] + v, 0.0)
    o_ref[0, :32, :] = hx.T
    o_ref[0, 32:, :] = x[:, :256].T


def _assemble(nfu, new_pad, wxt, bxr):
    bm = 512
    return pl.pallas_call(
        _asm_body,
        grid=(B, M // bm),
        in_specs=[
            pl.BlockSpec((1, bm, 288), lambda b, i: (b, i, 0)),
            pl.BlockSpec((1, bm, 4), lambda b, i: (b, i, 0)),
            pl.BlockSpec((4, 32), lambda b, i: (0, 0)),
            pl.BlockSpec((1, 32), lambda b, i: (0, 0)),
        ],
        out_specs=pl.BlockSpec((1, 288, bm), lambda b, i: (b, 0, i)),
        out_shape=jax.ShapeDtypeStruct((B, 288, M), jnp.float32),
    )(nfu, new_pad, wxt, bxr)


def kernel(xyz, new_xyz, features, W1, b1, W2, b2, Wxyz, bxyz):
    feats_t = features.transpose(0, 2, 1)
    xyz_pad = jnp.pad(xyz, ((0, 0), (0, 0), (0, 1)))
    new_pad = jnp.pad(new_xyz, ((0, 0), (0, 0), (0, 1)))
    xyz_t = xyz.transpose(0, 2, 1).reshape(-1)
    new_t = new_xyz.transpose(0, 2, 1).reshape(-1)
    wxt = jnp.pad(Wxyz.T, ((0, 1), (0, 0)))

    table = _mlp_table(feats_t, xyz_pad, W1.T, b1[None], W2.T, b2[None], wxt)
    idxg, wsel = _ballq(xyz_t, new_t)
    nfu = _aggregate(table.reshape(B * N, TW), idxg, wsel)
    out = _assemble(nfu.reshape(B, M, 288), new_pad, wxt, bxyz[None])
    return (new_xyz, out)


# speedup vs baseline: 20.2023x; 1.1483x over previous
import dataclasses
import functools

import jax
import jax.numpy as jnp
from jax import lax
from jax.experimental import pallas as pl
from jax.experimental.pallas import tpu as pltpu
from jax.experimental.pallas import tpu_sc as plsc

B, N, M, C = 4, 8192, 2048, 128
NS = 32
RADIUS2 = 0.2 * 0.2
NC, NSUB, L = 2, 16, 16
NTILE = NC * NSUB
QT = (B * M) // NTILE
TW = 384
HITCAP = 160
CHUNK = 8


def _sc_compiler_params():
    cp = pltpu.CompilerParams()
    if "needs_layout_passes" in pltpu.CompilerParams.__dataclass_fields__:
        cp = dataclasses.replace(cp, needs_layout_passes=False)
    return cp


def _mlp_body(f_ref, x_ref, w1_ref, b1_ref, w2_ref, b2_ref, wx_ref, t_ref):
    f = f_ref[0]
    h = jnp.maximum(
        jnp.dot(f, w1_ref[...], preferred_element_type=jnp.float32,
                precision=lax.Precision.HIGHEST)
        + b1_ref[...], 0.0)
    h = jnp.maximum(
        jnp.dot(h, w2_ref[...], preferred_element_type=jnp.float32,
                precision=lax.Precision.HIGHEST)
        + b2_ref[...], 0.0)
    u = jnp.dot(x_ref[0], wx_ref[...], preferred_element_type=jnp.float32,
                precision=lax.Precision.HIGHEST)
    t_ref[0, :, :256] = h
    t_ref[0, :, 256:288] = u
    t_ref[0, :, 288:] = jnp.zeros_like(t_ref[0, :, 288:])


def _mlp_table(feats_t, xyz_pad, w1t, b1r, w2t, b2r, wxt):
    bn = 512
    return pl.pallas_call(
        _mlp_body,
        grid=(B, N // bn),
        in_specs=[
            pl.BlockSpec((1, bn, C), lambda b, i: (b, i, 0)),
            pl.BlockSpec((1, bn, 4), lambda b, i: (b, i, 0)),
            pl.BlockSpec((C, 256), lambda b, i: (0, 0)),
            pl.BlockSpec((1, 256), lambda b, i: (0, 0)),
            pl.BlockSpec((256, 256), lambda b, i: (0, 0)),
            pl.BlockSpec((1, 256), lambda b, i: (0, 0)),
            pl.BlockSpec((4, 32), lambda b, i: (0, 0)),
        ],
        out_specs=pl.BlockSpec((1, bn, TW), lambda b, i: (b, i, 0)),
        out_shape=jax.ShapeDtypeStruct((B, N, TW), jnp.float32),
    )(feats_t, xyz_pad, w1t, b1r, w2t, b2r, wxt)


def _bf16_rne(x):
    i = plsc.bitcast(x, jnp.int32)
    r = (i + jnp.int32(0x7FFF) + ((i >> 16) & 1)) & jnp.int32(-65536)
    return plsc.bitcast(r, jnp.float32)


def _newton_rsqrt(s):
    i = plsc.bitcast(s, jnp.int32)
    y = plsc.bitcast(jnp.int32(0x5F3759DF) - (i >> 1), jnp.float32)
    for _ in range(3):
        y = y * (1.5 - 0.5 * s * y * y)
    return y


def _ballq_body(xyz_hbm, new_hbm, idx_hbm, w_hbm,
                xv, yv, zv, xbv, ybv, zbv, x2v, qxv, qyv, qzv, ibuf,
                oidx, ow):
    wid = lax.axis_index("s") * NC + lax.axis_index("c")
    q0 = wid * QT
    b = q0 // M
    m0 = q0 - b * M
    pltpu.sync_copy(xyz_hbm.at[pl.ds((b * 3 + 0) * N, N)], xv)
    pltpu.sync_copy(xyz_hbm.at[pl.ds((b * 3 + 1) * N, N)], yv)
    pltpu.sync_copy(xyz_hbm.at[pl.ds((b * 3 + 2) * N, N)], zv)
    pltpu.sync_copy(new_hbm.at[pl.ds((b * 3 + 0) * M + m0, QT)], qxv)
    pltpu.sync_copy(new_hbm.at[pl.ds((b * 3 + 1) * M + m0, QT)], qyv)
    pltpu.sync_copy(new_hbm.at[pl.ds((b * 3 + 2) * M + m0, QT)], qzv)

    lane = lax.iota(jnp.int32, L)
    zeros16 = jnp.zeros((L,), jnp.int32)

    @pl.loop(0, N // L)
    def _pre(i):
        base = i * L
        x = xv[pl.ds(base, L)]
        y = yv[pl.ds(base, L)]
        z = zv[pl.ds(base, L)]
        xbv[pl.ds(base, L)] = _bf16_rne(x)
        ybv[pl.ds(base, L)] = _bf16_rne(y)
        zbv[pl.ds(base, L)] = _bf16_rne(z)
        x2v[pl.ds(base, L)] = x * x + y * y + z * z

    @pl.loop(0, QT)
    def _per_query(j):
        jj = jnp.full((L,), j, jnp.int32)
        qx = plsc.load_gather(qxv, [jj])
        qy = plsc.load_gather(qyv, [jj])
        qz = plsc.load_gather(qzv, [jj])
        q2 = qx * qx + qy * qy + qz * qz
        qxb = _bf16_rne(qx)
        qyb = _bf16_rne(qy)
        qzb = _bf16_rne(qz)

        def cond(carry):
            n, cnt = carry
            return (n < N) & jnp.all(cnt < NS)

        def body(carry):
            n, cnt = carry
            for v in range(CHUNK):
                base = n + v * L
                dot = qxb * xbv[pl.ds(base, L)] \
                    + qyb * ybv[pl.ds(base, L)] \
                    + qzb * zbv[pl.ds(base, L)]
                d2s = (q2 + x2v[pl.ds(base, L)]) - 2.0 * dot
                hit = d2s < RADIUS2
                pos = cnt + plsc.cumsum(hit.astype(jnp.int32)) - 1
                plsc.store_scatter(ibuf, [pos], base + lane, mask=hit)
                cnt = cnt + plsc.all_reduce_population_count(hit)
            return n + CHUNK * L, cnt

        _, cnt_vec = lax.while_loop(cond, body, (jnp.int32(0), zeros16))
        cnt = jnp.minimum(jnp.min(cnt_vec), NS)
        cntv = jnp.full((L,), cnt, jnp.int32)
        nonempty = cntv > 0
        dupf = (NS + 1 - cntv).astype(jnp.float32)
        first_i = plsc.load_gather(ibuf, [zeros16])
        first_i = jnp.where(nonempty, first_i, 0)
        rs_tot = jnp.zeros((L,), jnp.float32)
        halves = []
        for h in range(2):
            k = lane + h * L
            valid = k < cntv
            idx_h = jnp.where(valid, ibuf[pl.ds(h * L, L)], first_i)
            idx_h = jnp.where(nonempty, idx_h, 0)
            dx = plsc.load_gather(xv, [idx_h]) - qx
            dy = plsc.load_gather(yv, [idx_h]) - qy
            dz = plsc.load_gather(zv, [idx_h]) - qz
            d2_h = dx * dx + dy * dy + dz * dz
            t = _newton_rsqrt(d2_h + 1e-12)
            r = t / (1.0 + 1e-8 * t)
            idn = jnp.where((k >= 1) & valid, 1.0, dupf)
            r = r / idn
            rs_tot = rs_tot + r
            halves.append((idx_h, r))
        norm = jnp.sum(rs_tot)
        for h, (idx_h, r) in enumerate(halves):
            w = jnp.where(nonempty, r / norm, 0.0)
            oidx[pl.ds(j * NS + h * L, L)] = idx_h + b * N
            ow[pl.ds(j * NS + h * L, L)] = w

    pltpu.sync_copy(oidx, idx_hbm.at[pl.ds(q0 * NS, QT * NS)])
    pltpu.sync_copy(ow, w_hbm.at[pl.ds(q0 * NS, QT * NS)])


def _ballq(xyz_t, new_t):
    mesh = plsc.VectorSubcoreMesh(core_axis_name="c", subcore_axis_name="s")
    kern = pl.kernel(
        _ballq_body,
        out_type=(
            jax.ShapeDtypeStruct((B * M * NS,), jnp.int32),
            jax.ShapeDtypeStruct((B * M * NS,), jnp.float32),
        ),
        mesh=mesh,
        compiler_params=_sc_compiler_params(),
        scratch_types=[
            pltpu.VMEM((N,), jnp.float32),
            pltpu.VMEM((N,), jnp.float32),
            pltpu.VMEM((N,), jnp.float32),
            pltpu.VMEM((N,), jnp.float32),
            pltpu.VMEM((N,), jnp.float32),
            pltpu.VMEM((N,), jnp.float32),
            pltpu.VMEM((N,), jnp.float32),
            pltpu.VMEM((QT,), jnp.float32),
            pltpu.VMEM((QT,), jnp.float32),
            pltpu.VMEM((QT,), jnp.float32),
            pltpu.VMEM((HITCAP,), jnp.int32),
            pltpu.VMEM((QT * NS,), jnp.int32),
            pltpu.VMEM((QT * NS,), jnp.float32),
        ],
    )
    return kern(xyz_t, new_t)


def _agg_body(t_hbm, idx_hbm, w_hbm, nfu_hbm, sidx, sw, hbuf0, hbuf1, onf,
              sem0, sem1):
    wid = lax.axis_index("s") * NC + lax.axis_index("c")
    q0 = wid * QT
    pltpu.sync_copy(idx_hbm.at[pl.ds(q0 * NS, QT * NS)], sidx)
    pltpu.sync_copy(w_hbm.at[pl.ds(q0 * NS, QT * NS)], sw)

    hbufs = [hbuf0, hbuf1]
    sems = [sem0, sem1]
    for rb in range(2):
        pltpu.async_copy(t_hbm.at[sidx.at[pl.ds(rb * NS, NS)]],
                         hbufs[rb], sems[rb])

    @pl.loop(0, QT, step=2)
    def _pair(j0):
        for rb in range(2):
            j = j0 + rb
            pltpu.make_async_copy(t_hbm.at[pl.ds(0, NS)],
                                  hbufs[rb], sems[rb]).wait()
            hbuf = hbufs[rb]
            accs = [jnp.zeros((L,), jnp.float32) for _ in range(16)]
            u0 = jnp.full((L,), -jnp.inf, jnp.float32)
            u1 = jnp.full((L,), -jnp.inf, jnp.float32)
            for k in range(NS):
                wk = plsc.load_gather(
                    sw, [jnp.full((L,), j * NS + k, jnp.int32)])
                for c in range(16):
                    accs[c] = accs[c] + wk * hbuf[k, pl.ds(c * L, L)]
                u0 = jnp.maximum(u0, hbuf[k, pl.ds(256, L)])
                u1 = jnp.maximum(u1, hbuf[k, pl.ds(256 + L, L)])
            for c in range(16):
                onf[pl.ds(j * 288 + c * L, L)] = accs[c]
            onf[pl.ds(j * 288 + 256, L)] = u0
            onf[pl.ds(j * 288 + 256 + L, L)] = u1
            jn = jnp.minimum(j + 2, QT - 1)
            pltpu.async_copy(t_hbm.at[sidx.at[pl.ds(jn * NS, NS)]],
                             hbufs[rb], sems[rb])

    for rb in range(2):
        pltpu.make_async_copy(t_hbm.at[pl.ds(0, NS)],
                              hbufs[rb], sems[rb]).wait()
    pltpu.sync_copy(onf, nfu_hbm.at[pl.ds(q0 * 288, QT * 288)])


def _aggregate(table, idxg, wsel):
    mesh = plsc.VectorSubcoreMesh(core_axis_name="c", subcore_axis_name="s")
    kern = pl.kernel(
        _agg_body,
        out_type=jax.ShapeDtypeStruct((B * M * 288,), jnp.float32),
        mesh=mesh,
        compiler_params=_sc_compiler_params(),
        scratch_types=[
            pltpu.VMEM((QT * NS,), jnp.int32),
            pltpu.VMEM((QT * NS,), jnp.float32),
            pltpu.VMEM((NS, TW), jnp.float32),
            pltpu.VMEM((NS, TW), jnp.float32),
            pltpu.VMEM((QT * 288,), jnp.float32),
            pltpu.SemaphoreType.DMA,
            pltpu.SemaphoreType.DMA,
        ],
    )
    return kern(table, idxg, wsel)


def _asm_body(nfu_ref, q_ref, wx_ref, bx_ref, o_ref):
    x = nfu_ref[0]
    v = bx_ref[...] - jnp.dot(q_ref[0], wx_ref[...],
                              preferred_element_type=jnp.float32,
                              precision=lax.Precision.HIGHEST)
    hx = jnp.maximum(x[:, 256:288] + v, 0.0)
    o_ref[0, :32, :] = hx.T
    o_ref[0, 32:, :] = x[:, :256].T


def _assemble(nfu, new_pad, wxt, bxr):
    bm = 512
    return pl.pallas_call(
        _asm_body,
        grid=(B, M // bm),
        in_specs=[
            pl.BlockSpec((1, bm, 288), lambda b, i: (b, i, 0)),
            pl.BlockSpec((1, bm, 4), lambda b, i: (b, i, 0)),
            pl.BlockSpec((4, 32), lambda b, i: (0, 0)),
            pl.BlockSpec((1, 32), lambda b, i: (0, 0)),
        ],
        out_specs=pl.BlockSpec((1, 288, bm), lambda b, i: (b, 0, i)),
        out_shape=jax.ShapeDtypeStruct((B, 288, M), jnp.float32),
    )(nfu, new_pad, wxt, bxr)


def kernel(xyz, new_xyz, features, W1, b1, W2, b2, Wxyz, bxyz):
    feats_t = features.transpose(0, 2, 1)
    xyz_pad = jnp.pad(xyz, ((0, 0), (0, 0), (0, 1)))
    new_pad = jnp.pad(new_xyz, ((0, 0), (0, 0), (0, 1)))
    xyz_t = xyz.transpose(0, 2, 1).reshape(-1)
    new_t = new_xyz.transpose(0, 2, 1).reshape(-1)
    wxt = jnp.pad(Wxyz.T, ((0, 1), (0, 0)))

    table = _mlp_table(feats_t, xyz_pad, W1.T, b1[None], W2.T, b2[None], wxt)
    idxg, wsel = _ballq(xyz_t, new_t)
    nfu = _aggregate(table.reshape(B * N, TW), idxg, wsel)
    out = _assemble(nfu.reshape(B, M, 288), new_pad, wxt, bxyz[None])
    return (new_xyz, out)


# speedup vs baseline: 20.2323x; 1.0015x over previous
import dataclasses
import functools

import jax
import jax.numpy as jnp
from jax import lax
from jax.experimental import pallas as pl
from jax.experimental.pallas import tpu as pltpu
from jax.experimental.pallas import tpu_sc as plsc

B, N, M, C = 4, 8192, 2048, 128
NS = 32
RADIUS2 = 0.2 * 0.2
NC, NSUB, L = 2, 16, 16
NTILE = NC * NSUB
QT = (B * M) // NTILE
TW = 384
HITCAP = 160
CHUNK = 8


def _sc_compiler_params():
    cp = pltpu.CompilerParams()
    if "needs_layout_passes" in pltpu.CompilerParams.__dataclass_fields__:
        cp = dataclasses.replace(cp, needs_layout_passes=False)
    return cp


def _mlp_body(f_ref, x_ref, w1_ref, b1_ref, w2_ref, b2_ref, wx_ref, t_ref):
    f = f_ref[0]
    h = jnp.maximum(
        jnp.dot(f, w1_ref[...], preferred_element_type=jnp.float32,
                precision=lax.Precision.HIGHEST)
        + b1_ref[...], 0.0)
    h = jnp.maximum(
        jnp.dot(h, w2_ref[...], preferred_element_type=jnp.float32,
                precision=lax.Precision.HIGHEST)
        + b2_ref[...], 0.0)
    u = jnp.dot(x_ref[0], wx_ref[...], preferred_element_type=jnp.float32,
                precision=lax.Precision.HIGHEST)
    t_ref[0, :, :256] = h
    t_ref[0, :, 256:288] = u
    t_ref[0, :, 288:] = jnp.zeros_like(t_ref[0, :, 288:])


def _mlp_table(feats_t, xyz_pad, w1t, b1r, w2t, b2r, wxt):
    bn = 512
    return pl.pallas_call(
        _mlp_body,
        grid=(B, N // bn),
        in_specs=[
            pl.BlockSpec((1, bn, C), lambda b, i: (b, i, 0)),
            pl.BlockSpec((1, bn, 4), lambda b, i: (b, i, 0)),
            pl.BlockSpec((C, 256), lambda b, i: (0, 0)),
            pl.BlockSpec((1, 256), lambda b, i: (0, 0)),
            pl.BlockSpec((256, 256), lambda b, i: (0, 0)),
            pl.BlockSpec((1, 256), lambda b, i: (0, 0)),
            pl.BlockSpec((4, 32), lambda b, i: (0, 0)),
        ],
        out_specs=pl.BlockSpec((1, bn, TW), lambda b, i: (b, i, 0)),
        out_shape=jax.ShapeDtypeStruct((B, N, TW), jnp.float32),
    )(feats_t, xyz_pad, w1t, b1r, w2t, b2r, wxt)


def _bf16_rne(x):
    i = plsc.bitcast(x, jnp.int32)
    r = (i + jnp.int32(0x7FFF) + ((i >> 16) & 1)) & jnp.int32(-65536)
    return plsc.bitcast(r, jnp.float32)


def _newton_rsqrt(s):
    i = plsc.bitcast(s, jnp.int32)
    y = plsc.bitcast(jnp.int32(0x5F3759DF) - (i >> 1), jnp.float32)
    for _ in range(3):
        y = y * (1.5 - 0.5 * s * y * y)
    return y


def _ballq_body(xyz_hbm, new_hbm, idx_hbm, w_hbm,
                xv, yv, zv, xbv, ybv, zbv, x2v, qxv, qyv, qzv, ibuf,
                oidx, ow):
    wid = lax.axis_index("s") * NC + lax.axis_index("c")
    q0 = wid * QT
    b = q0 // M
    m0 = q0 - b * M
    pltpu.sync_copy(xyz_hbm.at[pl.ds((b * 3 + 0) * N, N)], xv)
    pltpu.sync_copy(xyz_hbm.at[pl.ds((b * 3 + 1) * N, N)], yv)
    pltpu.sync_copy(xyz_hbm.at[pl.ds((b * 3 + 2) * N, N)], zv)
    pltpu.sync_copy(new_hbm.at[pl.ds((b * 3 + 0) * M + m0, QT)], qxv)
    pltpu.sync_copy(new_hbm.at[pl.ds((b * 3 + 1) * M + m0, QT)], qyv)
    pltpu.sync_copy(new_hbm.at[pl.ds((b * 3 + 2) * M + m0, QT)], qzv)

    lane = lax.iota(jnp.int32, L)
    zeros16 = jnp.zeros((L,), jnp.int32)

    @pl.loop(0, N // L)
    def _pre(i):
        base = i * L
        x = xv[pl.ds(base, L)]
        y = yv[pl.ds(base, L)]
        z = zv[pl.ds(base, L)]
        xbv[pl.ds(base, L)] = _bf16_rne(x)
        ybv[pl.ds(base, L)] = _bf16_rne(y)
        zbv[pl.ds(base, L)] = _bf16_rne(z)
        x2v[pl.ds(base, L)] = x * x + y * y + z * z

    @pl.loop(0, QT)
    def _per_query(j):
        jj = jnp.full((L,), j, jnp.int32)
        qx = plsc.load_gather(qxv, [jj])
        qy = plsc.load_gather(qyv, [jj])
        qz = plsc.load_gather(qzv, [jj])
        q2 = qx * qx + qy * qy + qz * qz
        qxb = _bf16_rne(qx)
        qyb = _bf16_rne(qy)
        qzb = _bf16_rne(qz)

        def cond(carry):
            n, cnt = carry
            return (n < N) & jnp.all(cnt < NS)

        def body(carry):
            n, cnt = carry
            for v in range(CHUNK):
                base = n + v * L
                dot = qxb * xbv[pl.ds(base, L)] \
                    + qyb * ybv[pl.ds(base, L)] \
                    + qzb * zbv[pl.ds(base, L)]
                d2s = (q2 + x2v[pl.ds(base, L)]) - 2.0 * dot
                hit = d2s < RADIUS2
                pos = cnt + plsc.cumsum(hit.astype(jnp.int32)) - 1
                plsc.store_scatter(ibuf, [pos], base + lane, mask=hit)
                cnt = cnt + plsc.all_reduce_population_count(hit)
            return n + CHUNK * L, cnt

        _, cnt_vec = lax.while_loop(cond, body, (jnp.int32(0), zeros16))
        cnt = jnp.minimum(jnp.min(cnt_vec), NS)
        cntv = jnp.full((L,), cnt, jnp.int32)
        nonempty = cntv > 0
        dupf = (NS + 1 - cntv).astype(jnp.float32)
        first_i = plsc.load_gather(ibuf, [zeros16])
        first_i = jnp.where(nonempty, first_i, 0)
        rs_tot = jnp.zeros((L,), jnp.float32)
        halves = []
        for h in range(2):
            k = lane + h * L
            valid = k < cntv
            idx_h = jnp.where(valid, ibuf[pl.ds(h * L, L)], first_i)
            idx_h = jnp.where(nonempty, idx_h, 0)
            dx = plsc.load_gather(xv, [idx_h]) - qx
            dy = plsc.load_gather(yv, [idx_h]) - qy
            dz = plsc.load_gather(zv, [idx_h]) - qz
            d2_h = dx * dx + dy * dy + dz * dz
            t = _newton_rsqrt(d2_h + 1e-12)
            r = t / (1.0 + 1e-8 * t)
            idn = jnp.where((k >= 1) & valid, 1.0, dupf)
            r = r / idn
            rs_tot = rs_tot + r
            halves.append((idx_h, r))
        norm = jnp.sum(rs_tot)
        for h, (idx_h, r) in enumerate(halves):
            w = jnp.where(nonempty, r / norm, 0.0)
            oidx[pl.ds(j * NS + h * L, L)] = idx_h + b * N
            ow[pl.ds(j * NS + h * L, L)] = w

    pltpu.sync_copy(oidx, idx_hbm.at[pl.ds(q0 * NS, QT * NS)])
    pltpu.sync_copy(ow, w_hbm.at[pl.ds(q0 * NS, QT * NS)])


def _ballq(xyz_t, new_t):
    mesh = plsc.VectorSubcoreMesh(core_axis_name="c", subcore_axis_name="s")
    kern = pl.kernel(
        _ballq_body,
        out_type=(
            jax.ShapeDtypeStruct((B * M * NS,), jnp.int32),
            jax.ShapeDtypeStruct((B * M * NS,), jnp.float32),
        ),
        mesh=mesh,
        compiler_params=_sc_compiler_params(),
        scratch_types=[
            pltpu.VMEM((N,), jnp.float32),
            pltpu.VMEM((N,), jnp.float32),
            pltpu.VMEM((N,), jnp.float32),
            pltpu.VMEM((N,), jnp.float32),
            pltpu.VMEM((N,), jnp.float32),
            pltpu.VMEM((N,), jnp.float32),
            pltpu.VMEM((N,), jnp.float32),
            pltpu.VMEM((QT,), jnp.float32),
            pltpu.VMEM((QT,), jnp.float32),
            pltpu.VMEM((QT,), jnp.float32),
            pltpu.VMEM((HITCAP,), jnp.int32),
            pltpu.VMEM((QT * NS,), jnp.int32),
            pltpu.VMEM((QT * NS,), jnp.float32),
        ],
    )
    return kern(xyz_t, new_t)


NRING = 4
OB = 32


def _agg_body(t_hbm, idx_hbm, w_hbm, nfu_hbm, sidx, sw,
              hbuf0, hbuf1, hbuf2, hbuf3, obuf,
              sem0, sem1, sem2, sem3):
    wid = lax.axis_index("s") * NC + lax.axis_index("c")
    q0 = wid * QT
    pltpu.sync_copy(idx_hbm.at[pl.ds(q0 * NS, QT * NS)], sidx)
    pltpu.sync_copy(w_hbm.at[pl.ds(q0 * NS, QT * NS)], sw)

    hbufs = [hbuf0, hbuf1, hbuf2, hbuf3]
    sems = [sem0, sem1, sem2, sem3]
    for rb in range(NRING):
        pltpu.async_copy(t_hbm.at[sidx.at[pl.ds(rb * NS, NS)]],
                         hbufs[rb], sems[rb])

    @pl.loop(0, QT, step=NRING)
    def _quad(j0):
        for rb in range(NRING):
            j = j0 + rb
            pltpu.make_async_copy(t_hbm.at[pl.ds(0, NS)],
                                  hbufs[rb], sems[rb]).wait()
            hbuf = hbufs[rb]
            accs = [jnp.zeros((L,), jnp.float32) for _ in range(16)]
            u0 = jnp.full((L,), -jnp.inf, jnp.float32)
            u1 = jnp.full((L,), -jnp.inf, jnp.float32)
            for k in range(NS):
                wk = plsc.load_gather(
                    sw, [jnp.full((L,), j * NS + k, jnp.int32)])
                for c in range(16):
                    accs[c] = accs[c] + wk * hbuf[k, pl.ds(c * L, L)]
                u0 = jnp.maximum(u0, hbuf[k, pl.ds(256, L)])
                u1 = jnp.maximum(u1, hbuf[k, pl.ds(256 + L, L)])
            o0 = (j & (OB - 1)) * 288
            for c in range(16):
                obuf[pl.ds(o0 + c * L, L)] = accs[c]
            obuf[pl.ds(o0 + 256, L)] = u0
            obuf[pl.ds(o0 + 256 + L, L)] = u1
            jn = jnp.minimum(j + NRING, QT - 1)
            pltpu.async_copy(t_hbm.at[sidx.at[pl.ds(jn * NS, NS)]],
                             hbufs[rb], sems[rb])

        @pl.when((j0 & (OB - 1)) == OB - NRING)
        def _flush():
            pltpu.sync_copy(
                obuf,
                nfu_hbm.at[pl.ds((q0 + j0 - (OB - NRING)) * 288, OB * 288)])

    for rb in range(NRING):
        pltpu.make_async_copy(t_hbm.at[pl.ds(0, NS)],
                              hbufs[rb], sems[rb]).wait()


def _aggregate(table, idxg, wsel):
    mesh = plsc.VectorSubcoreMesh(core_axis_name="c", subcore_axis_name="s")
    kern = pl.kernel(
        _agg_body,
        out_type=jax.ShapeDtypeStruct((B * M * 288,), jnp.float32),
        mesh=mesh,
        compiler_params=_sc_compiler_params(),
        scratch_types=[
            pltpu.VMEM((QT * NS,), jnp.int32),
            pltpu.VMEM((QT * NS,), jnp.float32),
            pltpu.VMEM((NS, TW), jnp.float32),
            pltpu.VMEM((NS, TW), jnp.float32),
            pltpu.VMEM((NS, TW), jnp.float32),
            pltpu.VMEM((NS, TW), jnp.float32),
            pltpu.VMEM((OB * 288,), jnp.float32),
            pltpu.SemaphoreType.DMA,
            pltpu.SemaphoreType.DMA,
            pltpu.SemaphoreType.DMA,
            pltpu.SemaphoreType.DMA,
        ],
    )
    return kern(table, idxg, wsel)


def _asm_body(nfu_ref, q_ref, wx_ref, bx_ref, o_ref):
    x = nfu_ref[0]
    v = bx_ref[...] - jnp.dot(q_ref[0], wx_ref[...],
                              preferred_element_type=jnp.float32,
                              precision=lax.Precision.HIGHEST)
    hx = jnp.maximum(x[:, 256:288] + v, 0.0)
    o_ref[0, :32, :] = hx.T
    o_ref[0, 32:, :] = x[:, :256].T


def _assemble(nfu, new_pad, wxt, bxr):
    bm = 512
    return pl.pallas_call(
        _asm_body,
        grid=(B, M // bm),
        in_specs=[
            pl.BlockSpec((1, bm, 288), lambda b, i: (b, i, 0)),
            pl.BlockSpec((1, bm, 4), lambda b, i: (b, i, 0)),
            pl.BlockSpec((4, 32), lambda b, i: (0, 0)),
            pl.BlockSpec((1, 32), lambda b, i: (0, 0)),
        ],
        out_specs=pl.BlockSpec((1, 288, bm), lambda b, i: (b, 0, i)),
        out_shape=jax.ShapeDtypeStruct((B, 288, M), jnp.float32),
    )(nfu, new_pad, wxt, bxr)


def kernel(xyz, new_xyz, features, W1, b1, W2, b2, Wxyz, bxyz):
    feats_t = features.transpose(0, 2, 1)
    xyz_pad = jnp.pad(xyz, ((0, 0), (0, 0), (0, 1)))
    new_pad = jnp.pad(new_xyz, ((0, 0), (0, 0), (0, 1)))
    xyz_t = xyz.transpose(0, 2, 1).reshape(-1)
    new_t = new_xyz.transpose(0, 2, 1).reshape(-1)
    wxt = jnp.pad(Wxyz.T, ((0, 1), (0, 0)))

    table = _mlp_table(feats_t, xyz_pad, W1.T, b1[None], W2.T, b2[None], wxt)
    idxg, wsel = _ballq(xyz_t, new_t)
    nfu = _aggregate(table.reshape(B * N, TW), idxg, wsel)
    out = _assemble(nfu.reshape(B, M, 288), new_pad, wxt, bxyz[None])
    return (new_xyz, out)


# speedup vs baseline: 34.8854x; 1.7242x over previous
import dataclasses
import functools

import jax
import jax.numpy as jnp
from jax import lax
from jax.experimental import pallas as pl
from jax.experimental.pallas import tpu as pltpu
from jax.experimental.pallas import tpu_sc as plsc

B, N, M, C = 4, 8192, 2048, 128
NS = 32
RADIUS2 = 0.2 * 0.2
NC, NSUB, L = 2, 16, 16
NTILE = NC * NSUB
QT = (B * M) // NTILE
TW = 384
HITCAP = 160
CHUNK = 8


def _sc_compiler_params():
    cp = pltpu.CompilerParams()
    if "needs_layout_passes" in pltpu.CompilerParams.__dataclass_fields__:
        cp = dataclasses.replace(cp, needs_layout_passes=False)
    return cp


def _mlp_body(f_ref, x_ref, w1_ref, b1_ref, w2_ref, b2_ref, wx_ref, t_ref):
    f = f_ref[0]
    h = jnp.maximum(
        jnp.dot(f, w1_ref[...], preferred_element_type=jnp.float32,
                precision=lax.Precision.HIGHEST)
        + b1_ref[...], 0.0)
    h = jnp.maximum(
        jnp.dot(h, w2_ref[...], preferred_element_type=jnp.float32,
                precision=lax.Precision.HIGHEST)
        + b2_ref[...], 0.0)
    u = jnp.dot(x_ref[0], wx_ref[...], preferred_element_type=jnp.float32,
                precision=lax.Precision.HIGHEST)
    t_ref[0, :, :256] = h
    t_ref[0, :, 256:288] = u
    t_ref[0, :, 288:] = jnp.zeros_like(t_ref[0, :, 288:])


def _mlp_table(feats_t, xyz_pad, w1t, b1r, w2t, b2r, wxt):
    bn = 512
    return pl.pallas_call(
        _mlp_body,
        grid=(B, N // bn),
        in_specs=[
            pl.BlockSpec((1, bn, C), lambda b, i: (b, i, 0)),
            pl.BlockSpec((1, bn, 4), lambda b, i: (b, i, 0)),
            pl.BlockSpec((C, 256), lambda b, i: (0, 0)),
            pl.BlockSpec((1, 256), lambda b, i: (0, 0)),
            pl.BlockSpec((256, 256), lambda b, i: (0, 0)),
            pl.BlockSpec((1, 256), lambda b, i: (0, 0)),
            pl.BlockSpec((4, 32), lambda b, i: (0, 0)),
        ],
        out_specs=pl.BlockSpec((1, bn, TW), lambda b, i: (b, i, 0)),
        out_shape=jax.ShapeDtypeStruct((B, N, TW), jnp.float32),
    )(feats_t, xyz_pad, w1t, b1r, w2t, b2r, wxt)


def _bf16_rne(x):
    i = plsc.bitcast(x, jnp.int32)
    r = (i + jnp.int32(0x7FFF) + ((i >> 16) & 1)) & jnp.int32(-65536)
    return plsc.bitcast(r, jnp.float32)


def _newton_rsqrt(s):
    i = plsc.bitcast(s, jnp.int32)
    y = plsc.bitcast(jnp.int32(0x5F3759DF) - (i >> 1), jnp.float32)
    for _ in range(3):
        y = y * (1.5 - 0.5 * s * y * y)
    return y


def _ballq_body(xyz_hbm, new_hbm, idx_hbm, w_hbm,
                xv, yv, zv, xbv, ybv, zbv, x2v, qxv, qyv, qzv, ibuf,
                oidx, ow):
    wid = lax.axis_index("s") * NC + lax.axis_index("c")
    q0 = wid * QT
    b = q0 // M
    m0 = q0 - b * M
    pltpu.sync_copy(xyz_hbm.at[pl.ds((b * 3 + 0) * N, N)], xv)
    pltpu.sync_copy(xyz_hbm.at[pl.ds((b * 3 + 1) * N, N)], yv)
    pltpu.sync_copy(xyz_hbm.at[pl.ds((b * 3 + 2) * N, N)], zv)
    pltpu.sync_copy(new_hbm.at[pl.ds((b * 3 + 0) * M + m0, QT)], qxv)
    pltpu.sync_copy(new_hbm.at[pl.ds((b * 3 + 1) * M + m0, QT)], qyv)
    pltpu.sync_copy(new_hbm.at[pl.ds((b * 3 + 2) * M + m0, QT)], qzv)

    lane = lax.iota(jnp.int32, L)
    zeros16 = jnp.zeros((L,), jnp.int32)

    @pl.loop(0, N // L)
    def _pre(i):
        base = i * L
        x = xv[pl.ds(base, L)]
        y = yv[pl.ds(base, L)]
        z = zv[pl.ds(base, L)]
        xbv[pl.ds(base, L)] = _bf16_rne(x)
        ybv[pl.ds(base, L)] = _bf16_rne(y)
        zbv[pl.ds(base, L)] = _bf16_rne(z)
        x2v[pl.ds(base, L)] = x * x + y * y + z * z

    @pl.loop(0, QT)
    def _per_query(j):
        jj = jnp.full((L,), j, jnp.int32)
        qx = plsc.load_gather(qxv, [jj])
        qy = plsc.load_gather(qyv, [jj])
        qz = plsc.load_gather(qzv, [jj])
        q2 = qx * qx + qy * qy + qz * qz
        qxb = _bf16_rne(qx)
        qyb = _bf16_rne(qy)
        qzb = _bf16_rne(qz)

        def cond(carry):
            n, cnt = carry
            return (n < N) & jnp.all(cnt < NS)

        def body(carry):
            n, cnt = carry
            for v in range(CHUNK):
                base = n + v * L
                dot = qxb * xbv[pl.ds(base, L)] \
                    + qyb * ybv[pl.ds(base, L)] \
                    + qzb * zbv[pl.ds(base, L)]
                d2s = (q2 + x2v[pl.ds(base, L)]) - 2.0 * dot
                hit = d2s < RADIUS2
                pos = cnt + plsc.cumsum(hit.astype(jnp.int32)) - 1
                plsc.store_scatter(ibuf, [pos], base + lane, mask=hit)
                cnt = cnt + plsc.all_reduce_population_count(hit)
            return n + CHUNK * L, cnt

        _, cnt_vec = lax.while_loop(cond, body, (jnp.int32(0), zeros16))
        cnt = jnp.minimum(jnp.min(cnt_vec), NS)
        cntv = jnp.full((L,), cnt, jnp.int32)
        nonempty = cntv > 0
        dupf = (NS + 1 - cntv).astype(jnp.float32)
        first_i = plsc.load_gather(ibuf, [zeros16])
        first_i = jnp.where(nonempty, first_i, 0)
        rs_tot = jnp.zeros((L,), jnp.float32)
        halves = []
        for h in range(2):
            k = lane + h * L
            valid = k < cntv
            idx_h = jnp.where(valid, ibuf[pl.ds(h * L, L)], first_i)
            idx_h = jnp.where(nonempty, idx_h, 0)
            dx = plsc.load_gather(xv, [idx_h]) - qx
            dy = plsc.load_gather(yv, [idx_h]) - qy
            dz = plsc.load_gather(zv, [idx_h]) - qz
            d2_h = dx * dx + dy * dy + dz * dz
            t = _newton_rsqrt(d2_h + 1e-12)
            r = t / (1.0 + 1e-8 * t)
            idn = jnp.where((k >= 1) & valid, 1.0, dupf)
            r = r / idn
            rs_tot = rs_tot + r
            halves.append((idx_h, r))
        norm = jnp.sum(rs_tot)
        for h, (idx_h, r) in enumerate(halves):
            w = jnp.where(nonempty, r / norm, 0.0)
            oidx[pl.ds(j * NS + h * L, L)] = idx_h + b * N
            ow[pl.ds(j * NS + h * L, L)] = w

    pltpu.sync_copy(oidx, idx_hbm.at[pl.ds(q0 * NS, QT * NS)])
    pltpu.sync_copy(ow, w_hbm.at[pl.ds(q0 * NS, QT * NS)])


def _ballq(xyz_t, new_t):
    mesh = plsc.VectorSubcoreMesh(core_axis_name="c", subcore_axis_name="s")
    kern = pl.kernel(
        _ballq_body,
        out_type=(
            jax.ShapeDtypeStruct((B * M * NS,), jnp.int32),
            jax.ShapeDtypeStruct((B * M * NS,), jnp.float32),
        ),
        mesh=mesh,
        compiler_params=_sc_compiler_params(),
        scratch_types=[
            pltpu.VMEM((N,), jnp.float32),
            pltpu.VMEM((N,), jnp.float32),
            pltpu.VMEM((N,), jnp.float32),
            pltpu.VMEM((N,), jnp.float32),
            pltpu.VMEM((N,), jnp.float32),
            pltpu.VMEM((N,), jnp.float32),
            pltpu.VMEM((N,), jnp.float32),
            pltpu.VMEM((QT,), jnp.float32),
            pltpu.VMEM((QT,), jnp.float32),
            pltpu.VMEM((QT,), jnp.float32),
            pltpu.VMEM((HITCAP,), jnp.int32),
            pltpu.VMEM((QT * NS,), jnp.int32),
            pltpu.VMEM((QT * NS,), jnp.float32),
        ],
    )
    return kern(xyz_t, new_t)


NRING = 4
OB = 32


def _agg_body(t_hbm, idx_hbm, w_hbm, nfu_hbm, sidx, sw,
              hbuf0, hbuf1, hbuf2, hbuf3, obuf,
              sem0, sem1, sem2, sem3):
    wid = lax.axis_index("s") * NC + lax.axis_index("c")
    q0 = wid * QT
    pltpu.sync_copy(idx_hbm.at[pl.ds(q0 * NS, QT * NS)], sidx)
    pltpu.sync_copy(w_hbm.at[pl.ds(q0 * NS, QT * NS)], sw)

    hbufs = [hbuf0, hbuf1, hbuf2, hbuf3]
    sems = [sem0, sem1, sem2, sem3]
    for rb in range(NRING):
        pltpu.async_copy(t_hbm.at[sidx.at[pl.ds(rb * NS, NS)]],
                         hbufs[rb], sems[rb])

    @pl.loop(0, QT, step=NRING)
    def _quad(j0):
        for rb in range(NRING):
            j = j0 + rb
            pltpu.make_async_copy(t_hbm.at[pl.ds(0, NS)],
                                  hbufs[rb], sems[rb]).wait()
            hbuf = hbufs[rb]

            def nb(k, carry):
                accs, u0, u1 = carry
                wk = plsc.load_gather(
                    sw, [jnp.full((L,), j * NS + k, jnp.int32)])
                accs = tuple(
                    accs[c] + wk * hbuf[k, pl.ds(c * L, L)]
                    for c in range(16))
                u0 = jnp.maximum(u0, hbuf[k, pl.ds(256, L)])
                u1 = jnp.maximum(u1, hbuf[k, pl.ds(256 + L, L)])
                return accs, u0, u1

            accs, u0, u1 = lax.fori_loop(
                0, NS, nb,
                (tuple(jnp.zeros((L,), jnp.float32) for _ in range(16)),
                 jnp.full((L,), -jnp.inf, jnp.float32),
                 jnp.full((L,), -jnp.inf, jnp.float32)))
            o0 = (j & (OB - 1)) * 288
            for c in range(16):
                obuf[pl.ds(o0 + c * L, L)] = accs[c]
            obuf[pl.ds(o0 + 256, L)] = u0
            obuf[pl.ds(o0 + 256 + L, L)] = u1
            jn = jnp.minimum(j + NRING, QT - 1)
            pltpu.async_copy(t_hbm.at[sidx.at[pl.ds(jn * NS, NS)]],
                             hbufs[rb], sems[rb])

        @pl.when((j0 & (OB - 1)) == OB - NRING)
        def _flush():
            pltpu.sync_copy(
                obuf,
                nfu_hbm.at[pl.ds((q0 + j0 - (OB - NRING)) * 288, OB * 288)])

    for rb in range(NRING):
        pltpu.make_async_copy(t_hbm.at[pl.ds(0, NS)],
                              hbufs[rb], sems[rb]).wait()


def _aggregate(table, idxg, wsel):
    mesh = plsc.VectorSubcoreMesh(core_axis_name="c", subcore_axis_name="s")
    kern = pl.kernel(
        _agg_body,
        out_type=jax.ShapeDtypeStruct((B * M * 288,), jnp.float32),
        mesh=mesh,
        compiler_params=_sc_compiler_params(),
        scratch_types=[
            pltpu.VMEM((QT * NS,), jnp.int32),
            pltpu.VMEM((QT * NS,), jnp.float32),
            pltpu.VMEM((NS, TW), jnp.float32),
            pltpu.VMEM((NS, TW), jnp.float32),
            pltpu.VMEM((NS, TW), jnp.float32),
            pltpu.VMEM((NS, TW), jnp.float32),
            pltpu.VMEM((OB * 288,), jnp.float32),
            pltpu.SemaphoreType.DMA,
            pltpu.SemaphoreType.DMA,
            pltpu.SemaphoreType.DMA,
            pltpu.SemaphoreType.DMA,
        ],
    )
    return kern(table, idxg, wsel)


def _asm_body(nfu_ref, q_ref, wx_ref, bx_ref, o_ref):
    x = nfu_ref[0]
    v = bx_ref[...] - jnp.dot(q_ref[0], wx_ref[...],
                              preferred_element_type=jnp.float32,
                              precision=lax.Precision.HIGHEST)
    hx = jnp.maximum(x[:, 256:288] + v, 0.0)
    o_ref[0, :32, :] = hx.T
    o_ref[0, 32:, :] = x[:, :256].T


def _assemble(nfu, new_pad, wxt, bxr):
    bm = 512
    return pl.pallas_call(
        _asm_body,
        grid=(B, M // bm),
        in_specs=[
            pl.BlockSpec((1, bm, 288), lambda b, i: (b, i, 0)),
            pl.BlockSpec((1, bm, 4), lambda b, i: (b, i, 0)),
            pl.BlockSpec((4, 32), lambda b, i: (0, 0)),
            pl.BlockSpec((1, 32), lambda b, i: (0, 0)),
        ],
        out_specs=pl.BlockSpec((1, 288, bm), lambda b, i: (b, 0, i)),
        out_shape=jax.ShapeDtypeStruct((B, 288, M), jnp.float32),
    )(nfu, new_pad, wxt, bxr)


def kernel(xyz, new_xyz, features, W1, b1, W2, b2, Wxyz, bxyz):
    feats_t = features.transpose(0, 2, 1)
    xyz_pad = jnp.pad(xyz, ((0, 0), (0, 0), (0, 1)))
    new_pad = jnp.pad(new_xyz, ((0, 0), (0, 0), (0, 1)))
    xyz_t = xyz.transpose(0, 2, 1).reshape(-1)
    new_t = new_xyz.transpose(0, 2, 1).reshape(-1)
    wxt = jnp.pad(Wxyz.T, ((0, 1), (0, 0)))

    table = _mlp_table(feats_t, xyz_pad, W1.T, b1[None], W2.T, b2[None], wxt)
    idxg, wsel = _ballq(xyz_t, new_t)
    nfu = _aggregate(table.reshape(B * N, TW), idxg, wsel)
    out = _assemble(nfu.reshape(B, M, 288), new_pad, wxt, bxyz[None])
    return (new_xyz, out)
